# baseline (speedup 1.0000x reference)
import jax
import jax.numpy as jnp
from jax import lax
from jax.experimental import pallas as pl
from jax.experimental.pallas import tpu as pltpu
from jax.experimental.pallas import tpu_sc as plsc

VOCAB_SIZE = 100000
MODEL_DIM = 1024
MAXLEN = 2048
BATCH = 4
SEQ = 2048

NUM_CORES = 2
NUM_SUBCORES = 16
NUM_WORKERS = NUM_CORES * NUM_SUBCORES
LANES = 16

SPW = SEQ // NUM_WORKERS
CHUNK = 16
NQ = SPW // CHUNK
NITER = NQ * BATCH
NBUF = 5
LOOKAHEAD = NBUF - 1


def _body(xr_hbm, tok_hbm, pos_hbm, out_hbm,
          pos0, pos1, idx_v, t0, t1, t2, t3, t4,
          g0, g1, g2, g3, g4, os0, os1, os2, os3, os4, ps0, ps1, isem):
    wid = lax.axis_index("s") * NUM_CORES + lax.axis_index("c")
    s0 = wid * SPW

    toks = (t0, t1, t2, t3, t4)
    gsems = (g0, g1, g2, g3, g4)
    osems = (os0, os1, os2, os3, os4)
    pbufs = (pos0, pos1)
    psems = (ps0, ps1)

    def start_pos(q):
        return pltpu.async_copy(
            pos_hbm.at[pl.ds(s0 + q * CHUNK, CHUNK)], pbufs[q % 2], psems[q % 2])

    pw0 = start_pos(0)

    iw = [
        pltpu.async_copy(
            xr_hbm.at[b, pl.ds(wid * NQ, NQ)], idx_v.at[pl.ds(b * NQ, NQ)],
            isem)
        for b in range(BATCH)
    ]

    def start_gather(it):
        q, b = divmod(it, BATCH)
        p = it % NBUF
        return pltpu.async_copy(
            tok_hbm.at[idx_v.at[b * NQ + q]], toks[p], gsems[p])

    HALF = CHUNK // 2

    def start_out_half(it, h):
        q, b = divmod(it, BATCH)
        p = it % NBUF
        return pltpu.async_copy(
            toks[p].at[pl.ds(h * HALF, HALF)],
            out_hbm.at[b, pl.ds(s0 + q * CHUNK + h * HALF, HALF)], osems[p])

    pw = [None] * NQ
    pw[0] = pw0

    g = [None] * NITER
    o = [None] * NITER
    for it in range(LOOKAHEAD):
        if it < BATCH:
            iw[it].wait()
        g[it] = start_gather(it)
    for b in range(min(LOOKAHEAD, BATCH), BATCH):
        iw[b].wait()

    for it in range(NITER):
        q, b = divmod(it, BATCH)
        p = it % NBUF
        if b == 0:
            pw[q].wait()
            if q + 1 < NQ:
                pw[q + 1] = start_pos(q + 1)
        g[it].wait()

        tok = toks[p]
        pos = pbufs[q % 2]

        def add_row(r, _, tok=tok, pos=pos):
            for j in range(MODEL_DIM // LANES):
                sl = pl.ds(j * LANES, LANES)
                tok[r, sl] = tok[r, sl] + pos[r, sl]
            return 0

        lax.fori_loop(0, HALF, add_row, 0)
        oa = start_out_half(it, 0)
        lax.fori_loop(HALF, CHUNK, add_row, 0)
        ob = start_out_half(it, 1)
        o[it] = (oa, ob)

        nxt = it + LOOKAHEAD
        if nxt < NITER:
            prev = nxt - NBUF
            if prev >= 0:
                o[prev][0].wait()
                o[prev][1].wait()
            g[nxt] = start_gather(nxt)

    for it in range(NITER - NBUF, NITER):
        if o[it] is not None:
            o[it][0].wait()
            o[it][1].wait()


@jax.jit
def _embed(xr, token_table, pos_table):
    mesh = plsc.VectorSubcoreMesh(core_axis_name="c", subcore_axis_name="s")
    return pl.kernel(
        _body,
        out_type=jax.ShapeDtypeStruct((BATCH, SEQ, MODEL_DIM), jnp.float32),
        mesh=mesh,
        scratch_types=(
            [pltpu.VMEM((CHUNK, MODEL_DIM), jnp.float32)] * 2
            + [pltpu.VMEM((NITER, CHUNK), jnp.int32)]
            + [pltpu.VMEM((CHUNK, MODEL_DIM), jnp.float32)] * NBUF
            + [pltpu.SemaphoreType.DMA] * (2 * NBUF + 3)
        ),
    )(xr, token_table, pos_table)


def kernel(x, token_table, pos_table):
    xr = x.astype(jnp.int32).reshape(BATCH, SEQ // CHUNK, CHUNK)
    return _embed(xr, token_table, pos_table)


# speedup vs baseline: 1.2822x; 1.2822x over previous
import jax
import jax.numpy as jnp
from jax import lax
from jax.experimental import pallas as pl
from jax.experimental.pallas import tpu as pltpu
from jax.experimental.pallas import tpu_sc as plsc

VOCAB_SIZE = 100000
MODEL_DIM = 1024
MAXLEN = 2048
BATCH = 4
SEQ = 2048

NUM_CORES = 2
NUM_SUBCORES = 16
NUM_WORKERS = NUM_CORES * NUM_SUBCORES
LANES = 16

SPW = SEQ // NUM_WORKERS
CHUNK = 16
NQ = SPW // CHUNK
NITER = NQ * BATCH
NBUF = 5
LOOKAHEAD = NBUF - 1


def _body(xr_hbm, tok_hbm, pos_hbm, out_hbm,
          pos0, pos1, idx_v, t0, t1, t2, t3, t4,
          g0, g1, g2, g3, g4, os0, os1, os2, os3, os4, ps0, ps1, isem):
    wid = lax.axis_index("s") * NUM_CORES + lax.axis_index("c")
    s0 = wid * SPW

    toks = (t0, t1, t2, t3, t4)
    gsems = (g0, g1, g2, g3, g4)
    osems = (os0, os1, os2, os3, os4)
    pbufs = (pos0, pos1)
    psems = (ps0, ps1)

    def start_pos(q):
        return pltpu.async_copy(
            pos_hbm.at[pl.ds(s0 + q * CHUNK, CHUNK)], pbufs[q % 2], psems[q % 2])

    pw0 = start_pos(0)

    iw = [
        pltpu.async_copy(
            xr_hbm.at[b, pl.ds(wid * NQ, NQ)], idx_v.at[pl.ds(b * NQ, NQ)],
            isem)
        for b in range(BATCH)
    ]

    def start_gather(it):
        q, b = divmod(it, BATCH)
        p = it % NBUF
        return pltpu.async_copy(
            tok_hbm.at[idx_v.at[b * NQ + q]], toks[p], gsems[p])

    def start_out(it):
        q, b = divmod(it, BATCH)
        p = it % NBUF
        return pltpu.async_copy(
            toks[p], out_hbm.at[b, pl.ds(s0 + q * CHUNK, CHUNK)], osems[p])

    pw = [None] * NQ
    pw[0] = pw0

    g = [None] * NITER
    o = [None] * NITER
    for it in range(LOOKAHEAD):
        if it < BATCH:
            iw[it].wait()
        g[it] = start_gather(it)
    for b in range(min(LOOKAHEAD, BATCH), BATCH):
        iw[b].wait()

    for it in range(NITER):
        q, b = divmod(it, BATCH)
        p = it % NBUF
        if b == 0:
            pw[q].wait()
            if q + 1 < NQ:
                pw[q + 1] = start_pos(q + 1)
        g[it].wait()

        tok = toks[p]
        pos = pbufs[q % 2]

        def add_row(r, _, tok=tok, pos=pos):
            for j in range(MODEL_DIM // LANES):
                sl = pl.ds(j * LANES, LANES)
                tok[r, sl] = tok[r, sl] + pos[r, sl]
            return 0

        lax.fori_loop(0, CHUNK, add_row, 0)
        o[it] = start_out(it)

        nxt = it + LOOKAHEAD
        if nxt < NITER:
            prev = nxt - NBUF
            if prev >= 0:
                o[prev].wait()
            g[nxt] = start_gather(nxt)

    for it in range(NITER - NBUF, NITER):
        if o[it] is not None:
            o[it].wait()


@jax.jit
def _embed(xr, token_table, pos_table):
    mesh = plsc.VectorSubcoreMesh(core_axis_name="c", subcore_axis_name="s")
    return pl.kernel(
        _body,
        out_type=jax.ShapeDtypeStruct((BATCH, SEQ, MODEL_DIM), jnp.float32),
        mesh=mesh,
        scratch_types=(
            [pltpu.VMEM((CHUNK, MODEL_DIM), jnp.float32)] * 2
            + [pltpu.VMEM((NITER, CHUNK), jnp.int32)]
            + [pltpu.VMEM((CHUNK, MODEL_DIM), jnp.float32)] * NBUF
            + [pltpu.SemaphoreType.DMA] * (2 * NBUF + 3)
        ),
    )(xr, token_table, pos_table)


def kernel(x, token_table, pos_table):
    xr = x.astype(jnp.int32).reshape(BATCH, SEQ // CHUNK, CHUNK)
    return _embed(xr, token_table, pos_table)


# speedup vs baseline: 1.2959x; 1.0107x over previous
import jax
import jax.numpy as jnp
from jax import lax
from jax.experimental import pallas as pl
from jax.experimental.pallas import tpu as pltpu
from jax.experimental.pallas import tpu_sc as plsc

VOCAB_SIZE = 100000
MODEL_DIM = 1024
MAXLEN = 2048
BATCH = 4
SEQ = 2048

NUM_CORES = 2
NUM_SUBCORES = 16
NUM_WORKERS = NUM_CORES * NUM_SUBCORES
LANES = 16

SPW = SEQ // NUM_WORKERS
CHUNK = 16
NQ = SPW // CHUNK
NITER = NQ * BATCH
NBUF = 5
LOOKAHEAD = NBUF - 2


def _body(xr_hbm, tok_hbm, pos_hbm, out_hbm,
          pos0, pos1, idx_v, t0, t1, t2, t3, t4,
          g0, g1, g2, g3, g4, os0, os1, os2, os3, os4, ps0, ps1, isem):
    wid = lax.axis_index("s") * NUM_CORES + lax.axis_index("c")
    s0 = wid * SPW

    toks = (t0, t1, t2, t3, t4)
    gsems = (g0, g1, g2, g3, g4)
    osems = (os0, os1, os2, os3, os4)
    pbufs = (pos0, pos1)
    psems = (ps0, ps1)

    def start_pos(q):
        return pltpu.async_copy(
            pos_hbm.at[pl.ds(s0 + q * CHUNK, CHUNK)], pbufs[q % 2], psems[q % 2])

    pw0 = start_pos(0)

    iw = [
        pltpu.async_copy(
            xr_hbm.at[b, pl.ds(wid * NQ, NQ)], idx_v.at[pl.ds(b * NQ, NQ)],
            isem)
        for b in range(BATCH)
    ]

    def start_gather(it):
        q, b = divmod(it, BATCH)
        p = it % NBUF
        return pltpu.async_copy(
            tok_hbm.at[idx_v.at[b * NQ + q]], toks[p], gsems[p])

    def start_out(it):
        q, b = divmod(it, BATCH)
        p = it % NBUF
        return pltpu.async_copy(
            toks[p], out_hbm.at[b, pl.ds(s0 + q * CHUNK, CHUNK)], osems[p])

    pw = [None] * NQ
    pw[0] = pw0

    g = [None] * NITER
    o = [None] * NITER
    for it in range(LOOKAHEAD):
        if it < BATCH:
            iw[it].wait()
        g[it] = start_gather(it)
    for b in range(min(LOOKAHEAD, BATCH), BATCH):
        iw[b].wait()

    for it in range(NITER):
        q, b = divmod(it, BATCH)
        p = it % NBUF
        if b == 0:
            pw[q].wait()
            if q + 1 < NQ:
                pw[q + 1] = start_pos(q + 1)
        g[it].wait()

        tok = toks[p]
        pos = pbufs[q % 2]

        def add_row(r, _, tok=tok, pos=pos):
            for j in range(MODEL_DIM // LANES):
                sl = pl.ds(j * LANES, LANES)
                tok[r, sl] = tok[r, sl] + pos[r, sl]
            return 0

        lax.fori_loop(0, CHUNK, add_row, 0)
        o[it] = start_out(it)

        nxt = it + LOOKAHEAD
        if nxt < NITER:
            prev = nxt - NBUF
            if prev >= 0:
                o[prev].wait()
            g[nxt] = start_gather(nxt)

    for it in range(NITER - NBUF, NITER):
        if o[it] is not None:
            o[it].wait()


@jax.jit
def _embed(xr, token_table, pos_table):
    mesh = plsc.VectorSubcoreMesh(core_axis_name="c", subcore_axis_name="s")
    return pl.kernel(
        _body,
        out_type=jax.ShapeDtypeStruct((BATCH, SEQ, MODEL_DIM), jnp.float32),
        mesh=mesh,
        scratch_types=(
            [pltpu.VMEM((CHUNK, MODEL_DIM), jnp.float32)] * 2
            + [pltpu.VMEM((NITER, CHUNK), jnp.int32)]
            + [pltpu.VMEM((CHUNK, MODEL_DIM), jnp.float32)] * NBUF
            + [pltpu.SemaphoreType.DMA] * (2 * NBUF + 3)
        ),
    )(xr, token_table, pos_table)


def kernel(x, token_table, pos_table):
    xr = x.astype(jnp.int32).reshape(BATCH, SEQ // CHUNK, CHUNK)
    return _embed(xr, token_table, pos_table)
